# baseline (device time: 530820 ns/iter reference)
import functools
import os

import jax

os.makedirs("/tmp/jax_cache", exist_ok=True)
jax.config.update("jax_compilation_cache_dir", "/tmp/jax_cache")
jax.config.update("jax_persistent_cache_min_compile_time_secs", 0.0)
jax.config.update("jax_persistent_cache_min_entry_size_bytes", -1)

import jax.numpy as jnp
import numpy as np
from jax import lax
from jax.experimental import pallas as pl
from jax.experimental.pallas import tpu as pltpu

N_DEV = 8
HEADS = 8
DH = 128
SQ = 1024
SKV = 1024
D = 1024
SCALE = 0.08838834764831843
NEG = -1e9

B0 = [0, 3, 6, 9, 12, 15]
B1 = [1, 4, 7, 10, 13]
B2 = [2, 5, 8, 11, 14]
C12 = [0, 1, 2, 4, 5, 7, 8, 10, 11, 13, 14]
N0, N1, N12 = 384, 320, 704

_PERM = np.concatenate([np.arange(b * 64, (b + 1) * 64) for b in B0 + B1 + B2])
_INV = np.argsort(_PERM)
_IDX0 = np.concatenate([np.arange(b * 64, (b + 1) * 64) for b in B0])
_IDX12 = np.concatenate([np.arange(b * 64, (b + 1) * 64) for b in C12])


def _bias(bands):
    qb = np.repeat(bands, 64)[:, None]
    kb = np.repeat(C12, 64)[None, :]
    keep = (qb == kb) | (kb == 0) | ((qb + kb) % 3 == 0)
    return np.where(keep, 0.0, NEG).astype(np.float32)


_M1 = _bias(B1)
_M2 = _bias(B2)


def _body(x_ref, wq_ref, k0_ref, k12_ref, v0_ref, v12_ref, wo_ref,
          m1_ref, m2_ref, out_ref,
          comm_ref, q_ref, ctx_ref, k0buf, k12buf, v0buf, v12buf,
          send_sems, recv_sems, kv_sems, credit_sem):
    my = lax.axis_index("i")
    left = jnp.mod(my - 1, N_DEV)
    right = jnp.mod(my + 1, N_DEV)

    def start_kv(slot, g):
        cps = []
        for j, (src, dst) in enumerate(
            [(k0_ref, k0buf), (k12_ref, k12buf),
             (v0_ref, v0buf), (v12_ref, v12buf)]):
            c = pltpu.make_async_copy(
                src.at[pl.ds(g * HEADS, HEADS)], dst.at[slot],
                kv_sems.at[slot, j])
            c.start()
            cps.append(c)
        return cps

    kv = start_kv(0, my)

    barrier = pltpu.get_barrier_semaphore()
    for nbr in (left, right):
        pl.semaphore_signal(barrier, inc=1, device_id=(nbr,),
                            device_id_type=pl.DeviceIdType.MESH)
    pl.semaphore_wait(barrier, 2)

    comm_ref[0, pl.ds(0, D), :] = wq_ref[...]
    comm_ref[0, pl.ds(D, D), :] = wo_ref[...]
    out_ref[0, :, :] = jnp.zeros((SQ, D), jnp.float32)

    for h in range(N_DEV):
        slot = h % 2
        g = jnp.mod(my - h, N_DEV)

        rdma = None
        if h < N_DEV - 1:
            if h >= 1:
                pl.semaphore_wait(credit_sem, 1)
            rdma = pltpu.make_async_remote_copy(
                src_ref=comm_ref.at[slot],
                dst_ref=comm_ref.at[1 - slot],
                send_sem=send_sems.at[slot],
                recv_sem=recv_sems.at[1 - slot],
                device_id=(right,),
                device_id_type=pl.DeviceIdType.MESH,
            )
            rdma.start()

        wq_g = comm_ref[slot, pl.ds(0, D), :]
        wo_g = comm_ref[slot, pl.ds(D, D), :]
        q_ref[...] = jax.lax.dot(
            x_ref[...], wq_g, preferred_element_type=jnp.float32
        ).astype(jnp.bfloat16)

        for c in kv:
            c.wait()
        if h < N_DEV - 1:
            kv = start_kv(1 - slot, jnp.mod(my - h - 1, N_DEV))

        def head_body(hh, carry, slot=slot):
            off = hh * DH

            def attend(row0, nrows, kh, vh, bias):
                q_c = q_ref[pl.ds(row0, nrows), pl.ds(off, DH)]
                s = lax.dot_general(
                    q_c, kh, (((1,), (1,)), ((), ())),
                    preferred_element_type=jnp.float32,
                ) * SCALE
                if bias is not None:
                    s = s + bias
                w = jnp.exp(s)
                r = 1.0 / jnp.sum(w, axis=1, keepdims=True)
                c = lax.dot_general(
                    w.astype(jnp.bfloat16), vh, (((1,), (0,)), ((), ())),
                    preferred_element_type=jnp.float32,
                )
                ctx_ref[pl.ds(row0, nrows), pl.ds(off, DH)] = (
                    (c * r).astype(jnp.bfloat16))

            k0h = k0buf[slot, pl.ds(hh, 1)][0]
            v0h = v0buf[slot, pl.ds(hh, 1)][0]
            k12h = k12buf[slot, pl.ds(hh, 1)][0]
            v12h = v12buf[slot, pl.ds(hh, 1)][0]

            attend(0, N0, k0h, v0h, None)
            attend(N0, N1, k12h, v12h, m1_ref[...])
            attend(N0 + N1, N1, k12h, v12h, m2_ref[...])
            return carry

        lax.fori_loop(0, HEADS, head_body, 0)

        out_ref[0, :, :] = out_ref[0, :, :] + jax.lax.dot(
            ctx_ref[...], wo_g, preferred_element_type=jnp.float32
        )

        if h <= N_DEV - 3:
            pl.semaphore_signal(credit_sem, inc=1, device_id=(left,),
                                device_id_type=pl.DeviceIdType.MESH)
        if rdma is not None:
            rdma.wait()

    @functools.partial(pl.run_scoped, exit_sem=pltpu.SemaphoreType.REGULAR)
    def _(exit_sem):
        for nbr in (left, right):
            pl.semaphore_signal(exit_sem, inc=1, device_id=(nbr,),
                                device_id_type=pl.DeviceIdType.MESH)
        pl.semaphore_wait(exit_sem, 2)


def kernel(x, Wq, K_ext, V_ext, Wo):
    i = lax.axis_index("i")
    xp = jnp.take(x[0], jnp.asarray(_PERM), axis=0).astype(jnp.bfloat16)
    wq = Wq.astype(jnp.bfloat16)
    wo = Wo.astype(jnp.bfloat16)
    k = lax.dynamic_index_in_dim(K_ext, i, 0, keepdims=False)
    v = lax.dynamic_index_in_dim(V_ext, i, 0, keepdims=False)
    kt = jnp.transpose(k, (1, 0, 2)).astype(jnp.bfloat16)
    vt = jnp.transpose(v, (1, 0, 2)).astype(jnp.bfloat16)
    k0 = jnp.take(kt, jnp.asarray(_IDX0), axis=1)
    v0 = jnp.take(vt, jnp.asarray(_IDX0), axis=1)
    k12 = jnp.take(kt, jnp.asarray(_IDX12), axis=1)
    v12 = jnp.take(vt, jnp.asarray(_IDX12), axis=1)

    vmem = functools.partial(pl.BlockSpec, memory_space=pltpu.VMEM)
    any_ = functools.partial(pl.BlockSpec, memory_space=pl.ANY)
    res = pl.pallas_call(
        _body,
        out_shape=jax.ShapeDtypeStruct((1, SQ, D), jnp.float32),
        in_specs=[vmem(), vmem(), any_(), any_(), any_(), any_(),
                  vmem(), vmem(), vmem()],
        out_specs=vmem(),
        scratch_shapes=[
            pltpu.VMEM((2, 2 * D, D), jnp.bfloat16),
            pltpu.VMEM((SQ, HEADS * DH), jnp.bfloat16),
            pltpu.VMEM((SQ, HEADS * DH), jnp.bfloat16),
            pltpu.VMEM((2, HEADS, N0, DH), jnp.bfloat16),
            pltpu.VMEM((2, HEADS, N12, DH), jnp.bfloat16),
            pltpu.VMEM((2, HEADS, N0, DH), jnp.bfloat16),
            pltpu.VMEM((2, HEADS, N12, DH), jnp.bfloat16),
            pltpu.SemaphoreType.DMA((2,)),
            pltpu.SemaphoreType.DMA((2,)),
            pltpu.SemaphoreType.DMA((2, 4)),
            pltpu.SemaphoreType.REGULAR,
        ],
        compiler_params=pltpu.CompilerParams(
            collective_id=0,
            vmem_limit_bytes=128 * 1024 * 1024,
        ),
    )(xp, wq, k0, k12, v0, v12, wo, jnp.asarray(_M1), jnp.asarray(_M2))
    return jnp.take(res[0], jnp.asarray(_INV), axis=0)[None]


# device time: 451893 ns/iter; 1.1747x vs baseline; 1.1747x over previous
import functools
import os

import jax

os.makedirs("/tmp/jax_cache", exist_ok=True)
jax.config.update("jax_compilation_cache_dir", "/tmp/jax_cache")
jax.config.update("jax_persistent_cache_min_compile_time_secs", 0.0)
jax.config.update("jax_persistent_cache_min_entry_size_bytes", -1)

import jax.numpy as jnp
import numpy as np
from jax import lax
from jax.experimental import pallas as pl
from jax.experimental.pallas import tpu as pltpu

N_DEV = 8
HEADS = 8
DH = 128
SQ = 1024
SKV = 1024
D = 1024
SCALE = 0.08838834764831843
NEG = -1e9

B0 = [0, 3, 6, 9, 12, 15]
B1 = [1, 4, 7, 10, 13]
B2 = [2, 5, 8, 11, 14]
C12 = [0, 1, 2, 4, 5, 7, 8, 10, 11, 13, 14]
N0, N1, N12 = 384, 320, 704

_PERM = np.concatenate([np.arange(b * 64, (b + 1) * 64) for b in B0 + B1 + B2])
_INV = np.argsort(_PERM)
_IDX0 = np.concatenate([np.arange(b * 64, (b + 1) * 64) for b in B0])
_IDX12 = np.concatenate([np.arange(b * 64, (b + 1) * 64) for b in C12])


def _bias(bands):
    qb = np.repeat(bands, 64)[:, None]
    kb = np.repeat(C12, 64)[None, :]
    keep = (qb == kb) | (kb == 0) | ((qb + kb) % 3 == 0)
    return np.where(keep, 0.0, NEG).astype(np.float32)


_M1 = _bias(B1)
_M2 = _bias(B2)


def _body(x_ref, wq_ref, k_ref, v_ref, wo_ref,
          m1_ref, m2_ref, out_ref,
          comm_ref, q_ref, ctx_ref, k0buf, k12buf, v0buf, v12buf,
          send_sems, recv_sems, kv_sems, credit_sem):
    my = lax.axis_index("i")
    left = jnp.mod(my - 1, N_DEV)
    right = jnp.mod(my + 1, N_DEV)

    def start_kv(slot, g):
        cps = []
        j = 0
        for src, dst, blocks in ((k_ref, k0buf, B0), (v_ref, v0buf, B0),
                                 (k_ref, k12buf, C12), (v_ref, v12buf, C12)):
            for bi, kb in enumerate(blocks):
                c = pltpu.make_async_copy(
                    src.at[pl.ds(g * HEADS, HEADS), pl.ds(kb * 64, 64)],
                    dst.at[slot, :, pl.ds(bi * 64, 64)],
                    kv_sems.at[slot, j])
                c.start()
                cps.append(c)
                j += 1
        return cps

    kv = start_kv(0, my)

    barrier = pltpu.get_barrier_semaphore()
    for nbr in (left, right):
        pl.semaphore_signal(barrier, inc=1, device_id=(nbr,),
                            device_id_type=pl.DeviceIdType.MESH)
    pl.semaphore_wait(barrier, 2)

    comm_ref[0, pl.ds(0, D), :] = wq_ref[...]
    comm_ref[0, pl.ds(D, D), :] = wo_ref[...]
    out_ref[0, :, :] = jnp.zeros((SQ, D), jnp.float32)

    for h in range(N_DEV):
        slot = h % 2
        g = jnp.mod(my - h, N_DEV)

        rdma = None
        if h < N_DEV - 1:
            if h >= 1:
                pl.semaphore_wait(credit_sem, 1)
            rdma = pltpu.make_async_remote_copy(
                src_ref=comm_ref.at[slot],
                dst_ref=comm_ref.at[1 - slot],
                send_sem=send_sems.at[slot],
                recv_sem=recv_sems.at[1 - slot],
                device_id=(right,),
                device_id_type=pl.DeviceIdType.MESH,
            )
            rdma.start()

        wq_g = comm_ref[slot, pl.ds(0, D), :]
        wo_g = comm_ref[slot, pl.ds(D, D), :]
        q_ref[...] = jax.lax.dot(
            x_ref[...], wq_g, preferred_element_type=jnp.float32
        ).astype(jnp.bfloat16)

        for c in kv:
            c.wait()
        if h < N_DEV - 1:
            kv = start_kv(1 - slot, jnp.mod(my - h - 1, N_DEV))

        def head_body(hh, carry, slot=slot):
            off = hh * DH

            def attend(row0, nrows, kh, vh, bias):
                q_c = q_ref[pl.ds(row0, nrows), pl.ds(off, DH)]
                s = lax.dot_general(
                    q_c, kh, (((1,), (1,)), ((), ())),
                    preferred_element_type=jnp.float32,
                ) * SCALE
                if bias is not None:
                    s = s + bias
                w = jnp.exp(s)
                r = 1.0 / jnp.sum(w, axis=1, keepdims=True)
                c = lax.dot_general(
                    w.astype(jnp.bfloat16), vh, (((1,), (0,)), ((), ())),
                    preferred_element_type=jnp.float32,
                )
                ctx_ref[pl.ds(row0, nrows), pl.ds(off, DH)] = (
                    (c * r).astype(jnp.bfloat16))

            k0h = k0buf[slot, pl.ds(hh, 1)][0]
            v0h = v0buf[slot, pl.ds(hh, 1)][0]
            k12h = k12buf[slot, pl.ds(hh, 1)][0]
            v12h = v12buf[slot, pl.ds(hh, 1)][0]

            attend(0, N0, k0h, v0h, None)
            attend(N0, N1, k12h, v12h, m1_ref[...])
            attend(N0 + N1, N1, k12h, v12h, m2_ref[...])
            return carry

        lax.fori_loop(0, HEADS, head_body, 0)

        out_ref[0, :, :] = out_ref[0, :, :] + jax.lax.dot(
            ctx_ref[...], wo_g, preferred_element_type=jnp.float32
        )

        if h <= N_DEV - 3:
            pl.semaphore_signal(credit_sem, inc=1, device_id=(left,),
                                device_id_type=pl.DeviceIdType.MESH)
        if rdma is not None:
            rdma.wait()

    @functools.partial(pl.run_scoped, exit_sem=pltpu.SemaphoreType.REGULAR)
    def _(exit_sem):
        for nbr in (left, right):
            pl.semaphore_signal(exit_sem, inc=1, device_id=(nbr,),
                                device_id_type=pl.DeviceIdType.MESH)
        pl.semaphore_wait(exit_sem, 2)


def kernel(x, Wq, K_ext, V_ext, Wo):
    i = lax.axis_index("i")
    xp = jnp.take(x[0], jnp.asarray(_PERM), axis=0).astype(jnp.bfloat16)
    wq = Wq.astype(jnp.bfloat16)
    wo = Wo.astype(jnp.bfloat16)
    k = lax.dynamic_index_in_dim(K_ext, i, 0, keepdims=False)
    v = lax.dynamic_index_in_dim(V_ext, i, 0, keepdims=False)
    kt = jnp.transpose(k, (1, 0, 2)).astype(jnp.bfloat16)
    vt = jnp.transpose(v, (1, 0, 2)).astype(jnp.bfloat16)

    vmem = functools.partial(pl.BlockSpec, memory_space=pltpu.VMEM)
    any_ = functools.partial(pl.BlockSpec, memory_space=pl.ANY)
    res = pl.pallas_call(
        _body,
        out_shape=jax.ShapeDtypeStruct((1, SQ, D), jnp.float32),
        in_specs=[vmem(), vmem(), any_(), any_(),
                  vmem(), vmem(), vmem()],
        out_specs=vmem(),
        scratch_shapes=[
            pltpu.VMEM((2, 2 * D, D), jnp.bfloat16),
            pltpu.VMEM((SQ, HEADS * DH), jnp.bfloat16),
            pltpu.VMEM((SQ, HEADS * DH), jnp.bfloat16),
            pltpu.VMEM((2, HEADS, N0, DH), jnp.bfloat16),
            pltpu.VMEM((2, HEADS, N12, DH), jnp.bfloat16),
            pltpu.VMEM((2, HEADS, N0, DH), jnp.bfloat16),
            pltpu.VMEM((2, HEADS, N12, DH), jnp.bfloat16),
            pltpu.SemaphoreType.DMA((2,)),
            pltpu.SemaphoreType.DMA((2,)),
            pltpu.SemaphoreType.DMA((2, 34)),
            pltpu.SemaphoreType.REGULAR,
        ],
        compiler_params=pltpu.CompilerParams(
            collective_id=0,
            vmem_limit_bytes=128 * 1024 * 1024,
        ),
    )(xp, wq, kt, vt, wo, jnp.asarray(_M1), jnp.asarray(_M2))
    return jnp.take(res[0], jnp.asarray(_INV), axis=0)[None]


# device time: 313034 ns/iter; 1.6957x vs baseline; 1.4436x over previous
import functools
import os

import jax

os.makedirs("/tmp/jax_cache", exist_ok=True)
jax.config.update("jax_compilation_cache_dir", "/tmp/jax_cache")
jax.config.update("jax_persistent_cache_min_compile_time_secs", 0.0)
jax.config.update("jax_persistent_cache_min_entry_size_bytes", -1)

import jax.numpy as jnp
import numpy as np
from jax import lax
from jax.experimental import pallas as pl
from jax.experimental.pallas import tpu as pltpu

N_DEV = 8
HEADS = 8
DH = 128
SQ = 1024
SKV = 1024
D = 1024
SCALE = 0.08838834764831843
NEG = -1e9

N_CW = 4
N_CCW = 3

B0 = [0, 3, 6, 9, 12, 15]
B1 = [1, 4, 7, 10, 13]
B2 = [2, 5, 8, 11, 14]
C12 = [0, 1, 2, 4, 5, 7, 8, 10, 11, 13, 14]
N0, N1, N12 = 384, 320, 704

_PERM = np.concatenate([np.arange(b * 64, (b + 1) * 64) for b in B0 + B1 + B2])
_INV = np.argsort(_PERM)


def _bias(bands):
    qb = np.repeat(bands, 64)[:, None]
    kb = np.repeat(C12, 64)[None, :]
    keep = (qb == kb) | (kb == 0) | ((qb + kb) % 3 == 0)
    return np.where(keep, 0.0, NEG).astype(np.float32)


_M1 = _bias(B1)
_M2 = _bias(B2)


def _body(x_ref, wq_ref, k_ref, v_ref, wo_ref, m1_ref, m2_ref, out_ref,
          cw_ref, ccw_ref, q_ref, ctx_ref,
          k0c, k12c, v0c, v12c, k0w, k12w, v0w, v12w,
          send_cw, recv_cw, send_ccw, recv_ccw,
          kvs_cw, kvs_ccw, credit_cw, credit_ccw):
    my = lax.axis_index("i")
    left = jnp.mod(my - 1, N_DEV)
    right = jnp.mod(my + 1, N_DEV)

    def start_kv(bufs, sems, slot, g):
        b0k, b12k, b0v, b12v = bufs
        cps = []
        j = 0
        for src, dst, blocks in ((k_ref, b0k, B0), (v_ref, b0v, B0),
                                 (k_ref, b12k, C12), (v_ref, b12v, C12)):
            for bi, kb in enumerate(blocks):
                c = pltpu.make_async_copy(
                    src.at[pl.ds(g * HEADS, HEADS), pl.ds(kb * 64, 64)],
                    dst.at[slot, :, pl.ds(bi * 64, 64)],
                    sems.at[slot, j])
                c.start()
                cps.append(c)
                j += 1
        return cps

    cwbufs = (k0c, k12c, v0c, v12c)
    ccwbufs = (k0w, k12w, v0w, v12w)

    kv_cw = start_kv(cwbufs, kvs_cw, 0, my)
    kv_ccw = None

    barrier = pltpu.get_barrier_semaphore()
    for nbr in (left, right):
        pl.semaphore_signal(barrier, inc=1, device_id=(nbr,),
                            device_id_type=pl.DeviceIdType.MESH)
    pl.semaphore_wait(barrier, 2)

    cw_ref[0, pl.ds(0, D), :] = wq_ref[...]
    cw_ref[0, pl.ds(D, D), :] = wo_ref[...]
    out_ref[0, :, :] = jnp.zeros((SQ, D), jnp.float32)

    def compute_group(cref, cslot, bufs, kslot):
        b0k, b12k, b0v, b12v = bufs
        wq_g = cref[cslot, pl.ds(0, D), :]
        wo_g = cref[cslot, pl.ds(D, D), :]
        q_ref[...] = jax.lax.dot(
            x_ref[...], wq_g, preferred_element_type=jnp.float32
        ).astype(jnp.bfloat16)

        def head_body(hh, carry):
            off = hh * DH

            def attend(row0, nrows, kh, vh, bias):
                q_c = q_ref[pl.ds(row0, nrows), pl.ds(off, DH)]
                s = lax.dot_general(
                    q_c, kh, (((1,), (1,)), ((), ())),
                    preferred_element_type=jnp.float32,
                ) * SCALE
                if bias is not None:
                    s = s + bias
                w = jnp.exp(s)
                r = 1.0 / jnp.sum(w, axis=1, keepdims=True)
                c = lax.dot_general(
                    w.astype(jnp.bfloat16), vh, (((1,), (0,)), ((), ())),
                    preferred_element_type=jnp.float32,
                )
                ctx_ref[pl.ds(row0, nrows), pl.ds(off, DH)] = (
                    (c * r).astype(jnp.bfloat16))

            k0h = b0k[kslot, pl.ds(hh, 1)][0]
            v0h = b0v[kslot, pl.ds(hh, 1)][0]
            k12h = b12k[kslot, pl.ds(hh, 1)][0]
            v12h = b12v[kslot, pl.ds(hh, 1)][0]

            attend(0, N0, k0h, v0h, None)
            attend(N0, N1, k12h, v12h, m1_ref[...])
            attend(N0 + N1, N1, k12h, v12h, m2_ref[...])
            return carry

        lax.fori_loop(0, HEADS, head_body, 0)

        out_ref[0, :, :] = out_ref[0, :, :] + jax.lax.dot(
            ctx_ref[...], wo_g, preferred_element_type=jnp.float32
        )

    for s in range(N_CW + 1):
        slot = s % 2

        rdma_cw = rdma_ccw = None
        if s < N_CW:
            if s >= 1:
                pl.semaphore_wait(credit_cw, 1)
            rdma_cw = pltpu.make_async_remote_copy(
                src_ref=cw_ref.at[slot],
                dst_ref=cw_ref.at[1 - slot],
                send_sem=send_cw.at[slot],
                recv_sem=recv_cw.at[1 - slot],
                device_id=(right,),
                device_id_type=pl.DeviceIdType.MESH,
            )
            rdma_cw.start()
        if s < N_CCW:
            if s == 2:
                pl.semaphore_wait(credit_ccw, 1)
            rdma_ccw = pltpu.make_async_remote_copy(
                src_ref=cw_ref.at[0] if s == 0 else ccw_ref.at[slot],
                dst_ref=ccw_ref.at[1 - slot],
                send_sem=send_ccw.at[slot],
                recv_sem=recv_ccw.at[1 - slot],
                device_id=(left,),
                device_id_type=pl.DeviceIdType.MESH,
            )
            rdma_ccw.start()

        for c in kv_cw:
            c.wait()
        if s < N_CW:
            kv_cw = start_kv(cwbufs, kvs_cw, 1 - slot, jnp.mod(my - s - 1, N_DEV))
        if s == 0:
            kv_ccw = start_kv(ccwbufs, kvs_ccw, 1, jnp.mod(my + 1, N_DEV))
        compute_group(cw_ref, slot, cwbufs, slot)
        if s <= N_CW - 2:
            pl.semaphore_signal(credit_cw, inc=1, device_id=(left,),
                                device_id_type=pl.DeviceIdType.MESH)

        if 1 <= s <= N_CCW:
            for c in kv_ccw:
                c.wait()
            if s < N_CCW:
                kv_ccw = start_kv(ccwbufs, kvs_ccw, 1 - slot,
                                  jnp.mod(my + s + 1, N_DEV))
            compute_group(ccw_ref, slot, ccwbufs, slot)
            if s == 1:
                pl.semaphore_signal(credit_ccw, inc=1, device_id=(right,),
                                    device_id_type=pl.DeviceIdType.MESH)

        if rdma_cw is not None:
            rdma_cw.wait()
        if rdma_ccw is not None:
            rdma_ccw.wait()

    @functools.partial(pl.run_scoped, exit_sem=pltpu.SemaphoreType.REGULAR)
    def _(exit_sem):
        for nbr in (left, right):
            pl.semaphore_signal(exit_sem, inc=1, device_id=(nbr,),
                                device_id_type=pl.DeviceIdType.MESH)
        pl.semaphore_wait(exit_sem, 2)


def kernel(x, Wq, K_ext, V_ext, Wo):
    i = lax.axis_index("i")
    xp = jnp.take(x[0], jnp.asarray(_PERM), axis=0).astype(jnp.bfloat16)
    wq = Wq.astype(jnp.bfloat16)
    wo = Wo.astype(jnp.bfloat16)
    k = lax.dynamic_index_in_dim(K_ext, i, 0, keepdims=False)
    v = lax.dynamic_index_in_dim(V_ext, i, 0, keepdims=False)
    kt = jnp.transpose(k, (1, 0, 2)).astype(jnp.bfloat16)
    vt = jnp.transpose(v, (1, 0, 2)).astype(jnp.bfloat16)

    vmem = functools.partial(pl.BlockSpec, memory_space=pltpu.VMEM)
    any_ = functools.partial(pl.BlockSpec, memory_space=pl.ANY)
    res = pl.pallas_call(
        _body,
        out_shape=jax.ShapeDtypeStruct((1, SQ, D), jnp.float32),
        in_specs=[vmem(), vmem(), any_(), any_(),
                  vmem(), vmem(), vmem()],
        out_specs=vmem(),
        scratch_shapes=[
            pltpu.VMEM((2, 2 * D, D), jnp.bfloat16),
            pltpu.VMEM((2, 2 * D, D), jnp.bfloat16),
            pltpu.VMEM((SQ, HEADS * DH), jnp.bfloat16),
            pltpu.VMEM((SQ, HEADS * DH), jnp.bfloat16),
            pltpu.VMEM((2, HEADS, N0, DH), jnp.bfloat16),
            pltpu.VMEM((2, HEADS, N12, DH), jnp.bfloat16),
            pltpu.VMEM((2, HEADS, N0, DH), jnp.bfloat16),
            pltpu.VMEM((2, HEADS, N12, DH), jnp.bfloat16),
            pltpu.VMEM((2, HEADS, N0, DH), jnp.bfloat16),
            pltpu.VMEM((2, HEADS, N12, DH), jnp.bfloat16),
            pltpu.VMEM((2, HEADS, N0, DH), jnp.bfloat16),
            pltpu.VMEM((2, HEADS, N12, DH), jnp.bfloat16),
            pltpu.SemaphoreType.DMA((2,)),
            pltpu.SemaphoreType.DMA((2,)),
            pltpu.SemaphoreType.DMA((2,)),
            pltpu.SemaphoreType.DMA((2,)),
            pltpu.SemaphoreType.DMA((2, 34)),
            pltpu.SemaphoreType.DMA((2, 34)),
            pltpu.SemaphoreType.REGULAR,
            pltpu.SemaphoreType.REGULAR,
        ],
        compiler_params=pltpu.CompilerParams(
            collective_id=0,
            vmem_limit_bytes=128 * 1024 * 1024,
        ),
    )(xp, wq, kt, vt, wo, jnp.asarray(_M1), jnp.asarray(_M2))
    return jnp.take(res[0], jnp.asarray(_INV), axis=0)[None]


# device time: 312966 ns/iter; 1.6961x vs baseline; 1.0002x over previous
import functools
import os

import jax

os.makedirs("/tmp/jax_cache", exist_ok=True)
jax.config.update("jax_compilation_cache_dir", "/tmp/jax_cache")
jax.config.update("jax_persistent_cache_min_compile_time_secs", 0.0)
jax.config.update("jax_persistent_cache_min_entry_size_bytes", -1)

import jax.numpy as jnp
import numpy as np
from jax import lax
from jax.experimental import pallas as pl
from jax.experimental.pallas import tpu as pltpu

N_DEV = 8
HEADS = 8
DH = 128
SQ = 1024
SKV = 1024
D = 1024
SCALE = 0.08838834764831843
NEG = -1e9

N_CW = 4
N_CCW = 3

B0 = [0, 3, 6, 9, 12, 15]
B1 = [1, 4, 7, 10, 13]
B2 = [2, 5, 8, 11, 14]
C12 = [0, 1, 2, 4, 5, 7, 8, 10, 11, 13, 14]
N0, N1, N12 = 384, 320, 704

_PERM = np.concatenate([np.arange(b * 64, (b + 1) * 64) for b in B0 + B1 + B2])
_INV = np.argsort(_PERM)


def _bias(bands):
    qb = np.repeat(bands, 64)[:, None]
    kb = np.repeat(C12, 64)[None, :]
    keep = (qb == kb) | (kb == 0) | ((qb + kb) % 3 == 0)
    return np.where(keep, 0.0, NEG).astype(np.float32)


_M12 = np.concatenate([_bias(B1), _bias(B2)], axis=0)


def _body(x_ref, wq_ref, k_ref, v_ref, wo_ref, m12_ref, out_ref,
          cw_ref, ccw_ref, q_ref, ctx_ref,
          k0c, k12c, v0c, v12c, k0w, k12w, v0w, v12w,
          send_cw, recv_cw, send_ccw, recv_ccw,
          kvs_cw, kvs_ccw, credit_cw, credit_ccw):
    my = lax.axis_index("i")
    left = jnp.mod(my - 1, N_DEV)
    right = jnp.mod(my + 1, N_DEV)

    def start_kv(bufs, sems, slot, g):
        b0k, b12k, b0v, b12v = bufs
        cps = []
        j = 0
        for src, dst, blocks in ((k_ref, b0k, B0), (v_ref, b0v, B0),
                                 (k_ref, b12k, C12), (v_ref, b12v, C12)):
            for bi, kb in enumerate(blocks):
                c = pltpu.make_async_copy(
                    src.at[pl.ds(g * HEADS, HEADS), pl.ds(kb * 64, 64)],
                    dst.at[slot, :, pl.ds(bi * 64, 64)],
                    sems.at[slot, j])
                c.start()
                cps.append(c)
                j += 1
        return cps

    cwbufs = (k0c, k12c, v0c, v12c)
    ccwbufs = (k0w, k12w, v0w, v12w)

    kv_cw = start_kv(cwbufs, kvs_cw, 0, my)
    kv_ccw = None

    barrier = pltpu.get_barrier_semaphore()
    for nbr in (left, right):
        pl.semaphore_signal(barrier, inc=1, device_id=(nbr,),
                            device_id_type=pl.DeviceIdType.MESH)
    pl.semaphore_wait(barrier, 2)

    cw_ref[0, pl.ds(0, D), :] = wq_ref[...]
    cw_ref[0, pl.ds(D, D), :] = wo_ref[...]
    out_ref[0, :, :] = jnp.zeros((SQ, D), jnp.float32)

    def compute_group(cref, cslot, bufs, kslot):
        b0k, b12k, b0v, b12v = bufs
        wq_g = cref[cslot, pl.ds(0, D), :]
        wo_g = cref[cslot, pl.ds(D, D), :]
        q_ref[...] = jax.lax.dot(
            x_ref[...], wq_g, preferred_element_type=jnp.float32
        ).astype(jnp.bfloat16)

        def head_body(hh, carry):
            off = hh * DH

            def attend(row0, nrows, kh, vh, bias):
                q_c = q_ref[pl.ds(row0, nrows), pl.ds(off, DH)]
                s = lax.dot_general(
                    q_c, kh, (((1,), (1,)), ((), ())),
                    preferred_element_type=jnp.float32,
                ) * SCALE
                if bias is not None:
                    s = s + bias
                w = jnp.exp(s)
                r = 1.0 / jnp.sum(w, axis=1, keepdims=True)
                c = lax.dot_general(
                    w.astype(jnp.bfloat16), vh, (((1,), (0,)), ((), ())),
                    preferred_element_type=jnp.float32,
                )
                ctx_ref[pl.ds(row0, nrows), pl.ds(off, DH)] = (
                    (c * r).astype(jnp.bfloat16))

            k0h = b0k[kslot, pl.ds(hh, 1)][0]
            v0h = b0v[kslot, pl.ds(hh, 1)][0]
            k12h = b12k[kslot, pl.ds(hh, 1)][0]
            v12h = b12v[kslot, pl.ds(hh, 1)][0]

            attend(0, N0, k0h, v0h, None)
            attend(N0, 2 * N1, k12h, v12h, m12_ref[...])
            return carry

        lax.fori_loop(0, HEADS, head_body, 0)

        out_ref[0, :, :] = out_ref[0, :, :] + jax.lax.dot(
            ctx_ref[...], wo_g, preferred_element_type=jnp.float32
        )

    for s in range(N_CW + 1):
        slot = s % 2

        rdma_cw = rdma_ccw = None
        if s < N_CW:
            if s >= 1:
                pl.semaphore_wait(credit_cw, 1)
            rdma_cw = pltpu.make_async_remote_copy(
                src_ref=cw_ref.at[slot],
                dst_ref=cw_ref.at[1 - slot],
                send_sem=send_cw.at[slot],
                recv_sem=recv_cw.at[1 - slot],
                device_id=(right,),
                device_id_type=pl.DeviceIdType.MESH,
            )
            rdma_cw.start()
        if s < N_CCW:
            if s == 2:
                pl.semaphore_wait(credit_ccw, 1)
            rdma_ccw = pltpu.make_async_remote_copy(
                src_ref=cw_ref.at[0] if s == 0 else ccw_ref.at[slot],
                dst_ref=ccw_ref.at[1 - slot],
                send_sem=send_ccw.at[slot],
                recv_sem=recv_ccw.at[1 - slot],
                device_id=(left,),
                device_id_type=pl.DeviceIdType.MESH,
            )
            rdma_ccw.start()

        for c in kv_cw:
            c.wait()
        if s < N_CW:
            kv_cw = start_kv(cwbufs, kvs_cw, 1 - slot, jnp.mod(my - s - 1, N_DEV))
        if s == 0:
            kv_ccw = start_kv(ccwbufs, kvs_ccw, 1, jnp.mod(my + 1, N_DEV))
        compute_group(cw_ref, slot, cwbufs, slot)
        if s <= N_CW - 2:
            pl.semaphore_signal(credit_cw, inc=1, device_id=(left,),
                                device_id_type=pl.DeviceIdType.MESH)

        if 1 <= s <= N_CCW:
            for c in kv_ccw:
                c.wait()
            if s < N_CCW:
                kv_ccw = start_kv(ccwbufs, kvs_ccw, 1 - slot,
                                  jnp.mod(my + s + 1, N_DEV))
            compute_group(ccw_ref, slot, ccwbufs, slot)
            if s == 1:
                pl.semaphore_signal(credit_ccw, inc=1, device_id=(right,),
                                    device_id_type=pl.DeviceIdType.MESH)

        if rdma_cw is not None:
            rdma_cw.wait()
        if rdma_ccw is not None:
            rdma_ccw.wait()

    @functools.partial(pl.run_scoped, exit_sem=pltpu.SemaphoreType.REGULAR)
    def _(exit_sem):
        for nbr in (left, right):
            pl.semaphore_signal(exit_sem, inc=1, device_id=(nbr,),
                                device_id_type=pl.DeviceIdType.MESH)
        pl.semaphore_wait(exit_sem, 2)


def kernel(x, Wq, K_ext, V_ext, Wo):
    i = lax.axis_index("i")
    xp = jnp.take(x[0], jnp.asarray(_PERM), axis=0).astype(jnp.bfloat16)
    wq = Wq.astype(jnp.bfloat16)
    wo = Wo.astype(jnp.bfloat16)
    k = lax.dynamic_index_in_dim(K_ext, i, 0, keepdims=False)
    v = lax.dynamic_index_in_dim(V_ext, i, 0, keepdims=False)
    kt = jnp.transpose(k, (1, 0, 2)).astype(jnp.bfloat16)
    vt = jnp.transpose(v, (1, 0, 2)).astype(jnp.bfloat16)

    vmem = functools.partial(pl.BlockSpec, memory_space=pltpu.VMEM)
    any_ = functools.partial(pl.BlockSpec, memory_space=pl.ANY)
    res = pl.pallas_call(
        _body,
        out_shape=jax.ShapeDtypeStruct((1, SQ, D), jnp.float32),
        in_specs=[vmem(), vmem(), any_(), any_(),
                  vmem(), vmem()],
        out_specs=vmem(),
        scratch_shapes=[
            pltpu.VMEM((2, 2 * D, D), jnp.bfloat16),
            pltpu.VMEM((2, 2 * D, D), jnp.bfloat16),
            pltpu.VMEM((SQ, HEADS * DH), jnp.bfloat16),
            pltpu.VMEM((SQ, HEADS * DH), jnp.bfloat16),
            pltpu.VMEM((2, HEADS, N0, DH), jnp.bfloat16),
            pltpu.VMEM((2, HEADS, N12, DH), jnp.bfloat16),
            pltpu.VMEM((2, HEADS, N0, DH), jnp.bfloat16),
            pltpu.VMEM((2, HEADS, N12, DH), jnp.bfloat16),
            pltpu.VMEM((2, HEADS, N0, DH), jnp.bfloat16),
            pltpu.VMEM((2, HEADS, N12, DH), jnp.bfloat16),
            pltpu.VMEM((2, HEADS, N0, DH), jnp.bfloat16),
            pltpu.VMEM((2, HEADS, N12, DH), jnp.bfloat16),
            pltpu.SemaphoreType.DMA((2,)),
            pltpu.SemaphoreType.DMA((2,)),
            pltpu.SemaphoreType.DMA((2,)),
            pltpu.SemaphoreType.DMA((2,)),
            pltpu.SemaphoreType.DMA((2, 34)),
            pltpu.SemaphoreType.DMA((2, 34)),
            pltpu.SemaphoreType.REGULAR,
            pltpu.SemaphoreType.REGULAR,
        ],
        compiler_params=pltpu.CompilerParams(
            collective_id=0,
            vmem_limit_bytes=128 * 1024 * 1024,
        ),
    )(xp, wq, kt, vt, wo, jnp.asarray(_M12))
    return jnp.take(res[0], jnp.asarray(_INV), axis=0)[None]


# device time: 306399 ns/iter; 1.7324x vs baseline; 1.0214x over previous
import functools
import os

import jax

os.makedirs("/tmp/jax_cache", exist_ok=True)
jax.config.update("jax_compilation_cache_dir", "/tmp/jax_cache")
jax.config.update("jax_persistent_cache_min_compile_time_secs", 0.0)
jax.config.update("jax_persistent_cache_min_entry_size_bytes", -1)

import jax.numpy as jnp
import numpy as np
from jax import lax
from jax.experimental import pallas as pl
from jax.experimental.pallas import tpu as pltpu

N_DEV = 8
HEADS = 8
DH = 128
SQ = 1024
SKV = 1024
D = 1024
SCALE = 0.08838834764831843
NEG = -1e9

N_CW = 4
N_CCW = 3

B0 = [0, 3, 6, 9, 12, 15]
B1 = [1, 4, 7, 10, 13]
B2 = [2, 5, 8, 11, 14]
C12 = [0, 1, 2, 4, 5, 7, 8, 10, 11, 13, 14]
N0, N1, N12 = 384, 320, 704

_BLKORDER = B0 + B1 + B2


def _bias(bands):
    qb = np.repeat(bands, 64)[:, None]
    kb = np.repeat(C12, 64)[None, :]
    keep = (qb == kb) | (kb == 0) | ((qb + kb) % 3 == 0)
    return np.where(keep, 0.0, NEG).astype(np.float32)


_M12 = np.concatenate([_bias(B1), _bias(B2)], axis=0)


def _body(x_ref, wq_ref, k_ref, v_ref, wo_ref, m12_ref, out_ref,
          cw_ref, ccw_ref, xp_ref, acc_ref, q_ref, ctx_ref,
          k0c, k12c, v0c, v12c, k0w, k12w, v0w, v12w,
          send_cw, recv_cw, send_ccw, recv_ccw,
          kvs_cw, kvs_ccw, credit_cw, credit_ccw):
    my = lax.axis_index("i")
    left = jnp.mod(my - 1, N_DEV)
    right = jnp.mod(my + 1, N_DEV)

    def start_kv(bufs, sems, slot, g):
        b0k, b12k, b0v, b12v = bufs
        cps = []
        j = 0
        for src, dst, blocks in ((k_ref, b0k, B0), (v_ref, b0v, B0),
                                 (k_ref, b12k, C12), (v_ref, b12v, C12)):
            for bi, kb in enumerate(blocks):
                c = pltpu.make_async_copy(
                    src.at[pl.ds(g * HEADS, HEADS), pl.ds(kb * 64, 64)],
                    dst.at[slot, :, pl.ds(bi * 64, 64)],
                    sems.at[slot, j])
                c.start()
                cps.append(c)
                j += 1
        return cps

    cwbufs = (k0c, k12c, v0c, v12c)
    ccwbufs = (k0w, k12w, v0w, v12w)

    kv_cw = start_kv(cwbufs, kvs_cw, 0, my)
    kv_ccw = None

    barrier = pltpu.get_barrier_semaphore()
    for nbr in (left, right):
        pl.semaphore_signal(barrier, inc=1, device_id=(nbr,),
                            device_id_type=pl.DeviceIdType.MESH)
    pl.semaphore_wait(barrier, 2)

    cw_ref[0, pl.ds(0, D), :] = wq_ref[...]
    cw_ref[0, pl.ds(D, D), :] = wo_ref[...]
    acc_ref[...] = jnp.zeros((SQ, D), jnp.float32)

    for j, b in enumerate(_BLKORDER):
        xp_ref[pl.ds(j * 64, 64), :] = x_ref[pl.ds(b * 64, 64), :]

    def compute_group(cref, cslot, bufs, kslot):
        b0k, b12k, b0v, b12v = bufs
        wq_g = cref[cslot, pl.ds(0, D), :]
        wo_g = cref[cslot, pl.ds(D, D), :]
        q_ref[...] = jax.lax.dot(
            xp_ref[...], wq_g, preferred_element_type=jnp.float32
        ).astype(jnp.bfloat16)

        def head_body(hh, carry):
            off = hh * DH

            def attend(row0, nrows, kh, vh, bias):
                q_c = q_ref[pl.ds(row0, nrows), pl.ds(off, DH)]
                s = lax.dot_general(
                    q_c, kh, (((1,), (1,)), ((), ())),
                    preferred_element_type=jnp.float32,
                ) * SCALE
                if bias is not None:
                    s = s + bias
                w = jnp.exp(s)
                r = 1.0 / jnp.sum(w, axis=1, keepdims=True)
                c = lax.dot_general(
                    w.astype(jnp.bfloat16), vh, (((1,), (0,)), ((), ())),
                    preferred_element_type=jnp.float32,
                )
                ctx_ref[pl.ds(row0, nrows), pl.ds(off, DH)] = (
                    (c * r).astype(jnp.bfloat16))

            k0h = b0k[kslot, pl.ds(hh, 1)][0]
            v0h = b0v[kslot, pl.ds(hh, 1)][0]
            k12h = b12k[kslot, pl.ds(hh, 1)][0]
            v12h = b12v[kslot, pl.ds(hh, 1)][0]

            attend(0, N0, k0h, v0h, None)
            attend(N0, 2 * N1, k12h, v12h, m12_ref[...])
            return carry

        lax.fori_loop(0, HEADS, head_body, 0)

        acc_ref[...] = acc_ref[...] + jax.lax.dot(
            ctx_ref[...], wo_g, preferred_element_type=jnp.float32
        )

    for s in range(N_CW + 1):
        slot = s % 2

        rdma_cw = rdma_ccw = None
        if s < N_CW:
            if s >= 1:
                pl.semaphore_wait(credit_cw, 1)
            rdma_cw = pltpu.make_async_remote_copy(
                src_ref=cw_ref.at[slot],
                dst_ref=cw_ref.at[1 - slot],
                send_sem=send_cw.at[slot],
                recv_sem=recv_cw.at[1 - slot],
                device_id=(right,),
                device_id_type=pl.DeviceIdType.MESH,
            )
            rdma_cw.start()
        if s < N_CCW:
            if s == 2:
                pl.semaphore_wait(credit_ccw, 1)
            rdma_ccw = pltpu.make_async_remote_copy(
                src_ref=cw_ref.at[0] if s == 0 else ccw_ref.at[slot],
                dst_ref=ccw_ref.at[1 - slot],
                send_sem=send_ccw.at[slot],
                recv_sem=recv_ccw.at[1 - slot],
                device_id=(left,),
                device_id_type=pl.DeviceIdType.MESH,
            )
            rdma_ccw.start()

        for c in kv_cw:
            c.wait()
        if s < N_CW:
            kv_cw = start_kv(cwbufs, kvs_cw, 1 - slot, jnp.mod(my - s - 1, N_DEV))
        if s == 0:
            kv_ccw = start_kv(ccwbufs, kvs_ccw, 1, jnp.mod(my + 1, N_DEV))
        compute_group(cw_ref, slot, cwbufs, slot)
        if s <= N_CW - 2:
            pl.semaphore_signal(credit_cw, inc=1, device_id=(left,),
                                device_id_type=pl.DeviceIdType.MESH)

        if 1 <= s <= N_CCW:
            for c in kv_ccw:
                c.wait()
            if s < N_CCW:
                kv_ccw = start_kv(ccwbufs, kvs_ccw, 1 - slot,
                                  jnp.mod(my + s + 1, N_DEV))
            compute_group(ccw_ref, slot, ccwbufs, slot)
            if s == 1:
                pl.semaphore_signal(credit_ccw, inc=1, device_id=(right,),
                                    device_id_type=pl.DeviceIdType.MESH)

        if rdma_cw is not None:
            rdma_cw.wait()
        if rdma_ccw is not None:
            rdma_ccw.wait()

    for j, b in enumerate(_BLKORDER):
        out_ref[0, pl.ds(b * 64, 64), :] = acc_ref[pl.ds(j * 64, 64), :]

    @functools.partial(pl.run_scoped, exit_sem=pltpu.SemaphoreType.REGULAR)
    def _(exit_sem):
        for nbr in (left, right):
            pl.semaphore_signal(exit_sem, inc=1, device_id=(nbr,),
                                device_id_type=pl.DeviceIdType.MESH)
        pl.semaphore_wait(exit_sem, 2)


def kernel(x, Wq, K_ext, V_ext, Wo):
    i = lax.axis_index("i")
    xs = x[0].astype(jnp.bfloat16)
    wq = Wq.astype(jnp.bfloat16)
    wo = Wo.astype(jnp.bfloat16)
    k = lax.dynamic_index_in_dim(K_ext, i, 0, keepdims=False)
    v = lax.dynamic_index_in_dim(V_ext, i, 0, keepdims=False)
    kt = jnp.transpose(k, (1, 0, 2)).astype(jnp.bfloat16)
    vt = jnp.transpose(v, (1, 0, 2)).astype(jnp.bfloat16)

    vmem = functools.partial(pl.BlockSpec, memory_space=pltpu.VMEM)
    any_ = functools.partial(pl.BlockSpec, memory_space=pl.ANY)
    res = pl.pallas_call(
        _body,
        out_shape=jax.ShapeDtypeStruct((1, SQ, D), jnp.float32),
        in_specs=[vmem(), vmem(), any_(), any_(),
                  vmem(), vmem()],
        out_specs=vmem(),
        scratch_shapes=[
            pltpu.VMEM((2, 2 * D, D), jnp.bfloat16),
            pltpu.VMEM((2, 2 * D, D), jnp.bfloat16),
            pltpu.VMEM((SQ, D), jnp.bfloat16),
            pltpu.VMEM((SQ, D), jnp.float32),
            pltpu.VMEM((SQ, HEADS * DH), jnp.bfloat16),
            pltpu.VMEM((SQ, HEADS * DH), jnp.bfloat16),
            pltpu.VMEM((2, HEADS, N0, DH), jnp.bfloat16),
            pltpu.VMEM((2, HEADS, N12, DH), jnp.bfloat16),
            pltpu.VMEM((2, HEADS, N0, DH), jnp.bfloat16),
            pltpu.VMEM((2, HEADS, N12, DH), jnp.bfloat16),
            pltpu.VMEM((2, HEADS, N0, DH), jnp.bfloat16),
            pltpu.VMEM((2, HEADS, N12, DH), jnp.bfloat16),
            pltpu.VMEM((2, HEADS, N0, DH), jnp.bfloat16),
            pltpu.VMEM((2, HEADS, N12, DH), jnp.bfloat16),
            pltpu.SemaphoreType.DMA((2,)),
            pltpu.SemaphoreType.DMA((2,)),
            pltpu.SemaphoreType.DMA((2,)),
            pltpu.SemaphoreType.DMA((2,)),
            pltpu.SemaphoreType.DMA((2, 34)),
            pltpu.SemaphoreType.DMA((2, 34)),
            pltpu.SemaphoreType.REGULAR,
            pltpu.SemaphoreType.REGULAR,
        ],
        compiler_params=pltpu.CompilerParams(
            collective_id=0,
            vmem_limit_bytes=128 * 1024 * 1024,
        ),
    )(xs, wq, kt, vt, wo, jnp.asarray(_M12))
    return res
